# baseline (speedup 1.0000x reference)
import functools

import jax
import jax.numpy as jnp
import numpy as np
from jax import lax
from jax.experimental import pallas as pl
from jax.experimental.pallas import tpu as pltpu
from jax.experimental.pallas import tpu_sc as plsc

_FIELD_DIMS = [100000] * 26
_NUM_FIELDS = len(_FIELD_DIMS)
_OFFSETS = np.concatenate(([0], np.cumsum(_FIELD_DIMS)[:-1])).astype(np.int32)

_B = 4096
_F = _NUM_FIELDS
_D = 32
_V = 100000 * 26
_NC, _NS = 2, 16
_NW = _NC * _NS
_BPW = _B // _NW
_NG = _BPW // 16
_SL = 512


def _fm_body(idxT_hbm, emb_hbm, lin_hbm, out_hbm,
             idxT_v, i4buf, gbuf, lgbuf, out_v, esem, lsem):
    w = lax.axis_index("s") * _NC + lax.axis_index("c")
    pltpu.sync_copy(idxT_hbm.at[w], idxT_v)

    zero = jnp.zeros((16,), jnp.float32)
    zero16i = jnp.zeros((16,), jnp.int32)

    for f in range(_F):
        pltpu.async_copy(
            lin_hbm.at[idxT_v.at[f]], lgbuf.at[pl.ds(f * 128, 128)], lsem)

    for b in range(2):
        for k in range(16 * _F // 16, _SL // 16):
            i4buf[b, pl.ds(k * 16, 16)] = zero16i

    def build(g, bb):
        def per_f(f, c2):
            i4buf[bb, pl.ds(f * 16, 16)] = idxT_v[f, pl.ds(g * 16, 16)]
            return c2
        lax.fori_loop(0, _F, per_f, 0)

    def fire(bb):
        def per_d(d, c2):
            for j in range(_SL // 128):
                pltpu.async_copy(
                    emb_hbm.at[d].at[i4buf.at[bb, pl.ds(j * 128, 128)]],
                    gbuf.at[bb, d, pl.ds(j * 128, 128)], esem)
            return c2
        lax.fori_loop(0, _D, per_d, 0)

    def drain(bb):
        pltpu.make_async_copy(
            emb_hbm.at[:, pl.ds(0, _SL)], gbuf.at[bb], esem).wait()

    def compute(g, bb):
        def per_d(d, ix):
            s = zero
            q = zero
            for f in range(_F):
                e = gbuf[bb, d, pl.ds(f * 16, 16)]
                s = s + e
                q = q + e * e
            return ix + (s * s - q)
        ix = lax.fori_loop(0, _D, per_d, zero)
        out_v[pl.ds(g * 16, 16)] = 0.5 * ix

    build(0, 0)
    fire(0)

    def per_group(g, carry):
        bb = g & 1

        @pl.when(g < _NG - 1)
        def _prefetch():
            build(g + 1, (g + 1) & 1)
            fire((g + 1) & 1)

        drain(bb)
        compute(g, bb)
        return carry

    lax.fori_loop(0, _NG, per_group, 0)

    pltpu.make_async_copy(
        lin_hbm.at[pl.ds(0, _F * 128)], lgbuf, lsem).wait()

    def add_lin(g, carry):
        def lin_f(f, acc):
            return acc + lgbuf[pl.ds(f * 128 + g * 16, 16)]
        ln = lax.fori_loop(0, _F, lin_f, zero)
        out_v[pl.ds(g * 16, 16)] = out_v[pl.ds(g * 16, 16)] + ln
        return carry

    lax.fori_loop(0, _NG, add_lin, 0)
    pltpu.sync_copy(out_v, out_hbm.at[w])


_fm_kernel = functools.partial(
    pl.kernel,
    out_type=jax.ShapeDtypeStruct((_NW, _BPW), jnp.float32),
    mesh=plsc.VectorSubcoreMesh(core_axis_name="c", subcore_axis_name="s"),
    scratch_types=[
        pltpu.VMEM((32, 128), jnp.int32),
        pltpu.VMEM((2, _SL), jnp.int32),
        pltpu.VMEM((2, _D, _SL), jnp.float32),
        pltpu.VMEM((_F * 128,), jnp.float32),
        pltpu.VMEM((_BPW,), jnp.float32),
        pltpu.SemaphoreType.DMA,
        pltpu.SemaphoreType.DMA,
    ],
    compiler_params=pltpu.CompilerParams(
        needs_layout_passes=False, use_tc_tiling_on_sc=False),
)(_fm_body)


def kernel(x, emb_table, linear_table, bias):
    offsets = jnp.asarray(_OFFSETS)
    idxT = x.T + offsets[:, None]
    idxT = jnp.pad(idxT, ((0, 32 - _F), (0, 0)))
    idxT3 = idxT.reshape(32, _NW, _BPW).transpose(1, 0, 2)
    lin_flat = linear_table.reshape(-1)
    embT = emb_table.T
    out = _fm_kernel(idxT3, embT, lin_flat)
    return out.reshape(_B, 1) + bias


# speedup vs baseline: 3.0913x; 3.0913x over previous
import functools

import jax
import jax.numpy as jnp
import numpy as np
from jax import lax
from jax.experimental import pallas as pl
from jax.experimental.pallas import tpu as pltpu
from jax.experimental.pallas import tpu_sc as plsc

_FIELD_DIMS = [100000] * 26
_NUM_FIELDS = len(_FIELD_DIMS)
_OFFSETS = np.concatenate(([0], np.cumsum(_FIELD_DIMS)[:-1])).astype(np.int32)

_B = 4096
_F = _NUM_FIELDS
_D = 32
_V = 100000 * 26
_V4 = _V // 4
_NC, _NS = 2, 16
_NW = _NC * _NS
_BPW = _B // _NW
_NG = _BPW // 16
_SL = 512


def _fm_body(idxT_hbm, tab4_hbm, lin_hbm, out_hbm,
             idxT_v, i4buf, gbuf, lgbuf, out_v, esem, lsem):
    w = lax.axis_index("s") * _NC + lax.axis_index("c")
    pltpu.sync_copy(idxT_hbm.at[w], idxT_v)

    zero = jnp.zeros((16,), jnp.float32)
    zero16i = jnp.zeros((16,), jnp.int32)
    lane = lax.iota(jnp.int32, 16)
    lane_f = lane * _F

    for f in range(_F):
        pltpu.async_copy(
            lin_hbm.at[idxT_v.at[f]], lgbuf.at[pl.ds(f * 128, 128)], lsem)

    for k in range(16 * _F // 16, _SL // 16):
        i4buf[0, pl.ds(k * 16, 16)] = zero16i

    def build(g, bb):
        def per_f(f, c2):
            rvec = idxT_v[f, pl.ds(g * 16, 16)]
            plsc.store_scatter(i4buf.at[bb], [lane_f + f], rvec >> 2)
            return c2
        lax.fori_loop(0, _F, per_f, 0)

    def fire(bb):
        for j in range(_SL // 128):
            pltpu.async_copy(
                tab4_hbm.at[i4buf.at[bb, pl.ds(j * 128, 128)]],
                gbuf.at[bb, pl.ds(j * 128, 128), :], esem)

    def drain(bb):
        pltpu.make_async_copy(
            tab4_hbm.at[pl.ds(0, _SL)], gbuf.at[bb], esem).wait()

    def compute(g, bb):
        def per_d(d, ix):
            s = zero
            q = zero
            for f in range(_F):
                rvec = idxT_v[f, pl.ds(g * 16, 16)]
                colb = (rvec & 3) << 5
                e = plsc.load_gather(gbuf.at[bb], [lane_f + f, colb + d])
                s = s + e
                q = q + e * e
            return ix + (s * s - q)
        ix = lax.fori_loop(0, _D, per_d, zero)
        out_v[pl.ds(g * 16, 16)] = 0.5 * ix

    def per_group(g, carry):
        build(g, 0)
        fire(0)
        drain(0)
        compute(g, 0)
        return carry

    lax.fori_loop(0, _NG, per_group, 0)

    pltpu.make_async_copy(
        lin_hbm.at[pl.ds(0, _F * 128)], lgbuf, lsem).wait()

    def add_lin(g, carry):
        def lin_f(f, acc):
            return acc + lgbuf[pl.ds(f * 128 + g * 16, 16)]
        ln = lax.fori_loop(0, _F, lin_f, zero)
        out_v[pl.ds(g * 16, 16)] = out_v[pl.ds(g * 16, 16)] + ln
        return carry

    lax.fori_loop(0, _NG, add_lin, 0)
    pltpu.sync_copy(out_v, out_hbm.at[w])


_fm_kernel = functools.partial(
    pl.kernel,
    out_type=jax.ShapeDtypeStruct((_NW, _BPW), jnp.float32),
    mesh=plsc.VectorSubcoreMesh(core_axis_name="c", subcore_axis_name="s"),
    scratch_types=[
        pltpu.VMEM((32, 128), jnp.int32),
        pltpu.VMEM((1, _SL), jnp.int32),
        pltpu.VMEM((1, _SL, 128), jnp.float32),
        pltpu.VMEM((_F * 128,), jnp.float32),
        pltpu.VMEM((_BPW,), jnp.float32),
        pltpu.SemaphoreType.DMA,
        pltpu.SemaphoreType.DMA,
    ],
    compiler_params=pltpu.CompilerParams(
        needs_layout_passes=False, use_tc_tiling_on_sc=False),
)(_fm_body)


def kernel(x, emb_table, linear_table, bias):
    offsets = jnp.asarray(_OFFSETS)
    idxT = x.T + offsets[:, None]
    idxT = jnp.pad(idxT, ((0, 32 - _F), (0, 0)))
    idxT3 = idxT.reshape(32, _NW, _BPW).transpose(1, 0, 2)
    lin_flat = linear_table.reshape(-1)
    tab4 = emb_table.reshape(_V4, 128)
    out = _fm_kernel(idxT3, tab4, lin_flat)
    return out.reshape(_B, 1) + bias


# speedup vs baseline: 4.5361x; 1.4674x over previous
import functools

import jax
import jax.numpy as jnp
import numpy as np
from jax import lax
from jax.experimental import pallas as pl
from jax.experimental.pallas import tpu as pltpu
from jax.experimental.pallas import tpu_sc as plsc

_FIELD_DIMS = [100000] * 26
_NUM_FIELDS = len(_FIELD_DIMS)
_OFFSETS = np.concatenate(([0], np.cumsum(_FIELD_DIMS)[:-1])).astype(np.int32)

_B = 4096
_F = _NUM_FIELDS
_FPAD = 32
_D = 32
_NC, _NS = 2, 16
_NW = _NC * _NS
_BPW = _B // _NW
_CHUNK = 128


def _fm_body(idx_hbm, idxp_hbm, emb_hbm, lin_hbm, out_hbm,
             idx_v, idxp_v, rows_v, lin_v, tvals_v, out_v, esem, lsem):
    w = lax.axis_index("s") * _NC + lax.axis_index("c")

    pltpu.sync_copy(idx_hbm.at[w], idx_v)
    pltpu.sync_copy(idxp_hbm.at[w], idxp_v)

    emb_copies = []
    for j in range(_F):
        c = pltpu.async_copy(
            emb_hbm.at[idx_v.at[j]], rows_v.at[pl.ds(j * _CHUNK, _CHUNK)], esem)
        emb_copies.append(c)
    lin_copies = []
    for j in range(_FPAD):
        c = pltpu.async_copy(
            lin_hbm.at[idxp_v.at[j]], lin_v.at[pl.ds(j * _CHUNK, _CHUNK)], lsem)
        lin_copies.append(c)
    for c in emb_copies:
        c.wait()
    for c in lin_copies:
        c.wait()

    lane = lax.iota(jnp.int32, 16)
    padmask = lane < (_F - 16)
    zero = jnp.zeros((16,), jnp.float32)

    def row_body(r, carry):
        s0 = zero
        s1 = zero
        q0 = zero
        q1 = zero
        for f in range(_F):
            eb = rows_v[r * _F + f, :]
            e0, e1 = plsc.unpack(eb, format=plsc.PackFormat.INTERLEAVED)
            s0 = s0 + e0
            q0 = q0 + e0 * e0
            s1 = s1 + e1
            q1 = q1 + e1 * e1
        t = (s0 * s0 - q0) + (s1 * s1 - q1)
        l0 = plsc.load_gather(lin_v, [r * _FPAD + lane])
        l1 = plsc.load_gather(lin_v, [r * _FPAD + 16 + lane])
        l1 = jnp.where(padmask, l1, 0.0)
        tvals_v[r, :] = 0.5 * t + l0 + l1
        return carry

    lax.fori_loop(0, _BPW, row_body, 0)

    for g in range(_BPW // 16):
        rows16 = g * 16 + lane
        acc = zero
        for d in range(16):
            acc = acc + plsc.load_gather(
                tvals_v, [rows16, jnp.full((16,), d, jnp.int32)])
        out_v[pl.ds(g * 16, 16)] = acc

    pltpu.sync_copy(out_v, out_hbm.at[w])


_fm_kernel = functools.partial(
    pl.kernel,
    out_type=jax.ShapeDtypeStruct((_NW, _BPW), jnp.float32),
    mesh=plsc.VectorSubcoreMesh(core_axis_name="c", subcore_axis_name="s"),
    scratch_types=[
        pltpu.VMEM((_F, _CHUNK), jnp.int32),
        pltpu.VMEM((_FPAD, _CHUNK), jnp.int32),
        pltpu.VMEM((_BPW * _F, _D), jnp.bfloat16),
        pltpu.VMEM((_BPW * _FPAD,), jnp.float32),
        pltpu.VMEM((_BPW, 16), jnp.float32),
        pltpu.VMEM((_BPW,), jnp.float32),
        pltpu.SemaphoreType.DMA,
        pltpu.SemaphoreType.DMA,
    ],
    compiler_params=pltpu.CompilerParams(
        needs_layout_passes=False, use_tc_tiling_on_sc=False),
)(_fm_body)


def kernel(x, emb_table, linear_table, bias):
    offsets = jnp.asarray(_OFFSETS)
    idx = x + offsets[None, :]
    idxp = jnp.concatenate([idx, idx[:, : _FPAD - _F]], axis=1)
    idx3 = idx.reshape(_NW, _F, _CHUNK)
    idxp3 = idxp.reshape(_NW, _FPAD, _CHUNK)
    lin_flat = linear_table.reshape(-1)
    emb16 = emb_table.astype(jnp.bfloat16)
    out = _fm_kernel(idx3, idxp3, emb16, lin_flat)
    return out.reshape(_B, 1) + bias


# speedup vs baseline: 5.3612x; 1.1819x over previous
import functools

import jax
import jax.numpy as jnp
import numpy as np
from jax import lax
from jax.experimental import pallas as pl
from jax.experimental.pallas import tpu as pltpu
from jax.experimental.pallas import tpu_sc as plsc

_FIELD_DIMS = [100000] * 26
_NUM_FIELDS = len(_FIELD_DIMS)
_OFFSETS = np.concatenate(([0], np.cumsum(_FIELD_DIMS)[:-1])).astype(np.int32)

_B = 4096
_F = _NUM_FIELDS
_FPAD = 32
_D = 32
_NC, _NS = 2, 16
_NW = _NC * _NS
_BPW = _B // _NW
_CHUNK = 128


def _fm_body(idx_hbm, idxp_hbm, emb_hbm, lin_hbm, out_hbm,
             idx_v, idxp_v, rows_v, lin_v, tvals_v, out_v, esem, lsem):
    w = lax.axis_index("s") * _NC + lax.axis_index("c")

    pltpu.sync_copy(idx_hbm.at[w], idx_v)
    pltpu.sync_copy(idxp_hbm.at[w], idxp_v)

    emb_copies = []
    for j in range(_F):
        c = pltpu.async_copy(
            emb_hbm.at[idx_v.at[j]], rows_v.at[pl.ds(j * _CHUNK, _CHUNK)], esem)
        emb_copies.append(c)
    lin_copies = []
    for j in range(_FPAD):
        c = pltpu.async_copy(
            lin_hbm.at[idxp_v.at[j]], lin_v.at[pl.ds(j * _CHUNK, _CHUNK)], lsem)
        lin_copies.append(c)
    for c in emb_copies:
        c.wait()
    for c in lin_copies:
        c.wait()

    lane = lax.iota(jnp.int32, 16)
    padmask = lane < (_F - 16)
    zero = jnp.zeros((16,), jnp.float32)

    def row_body(r, carry):
        s0 = zero
        s1 = zero
        q0 = zero
        q1 = zero
        for f in range(_F):
            e0 = rows_v[r * _F + f, pl.ds(0, 16)]
            e1 = rows_v[r * _F + f, pl.ds(16, 16)]
            s0 = s0 + e0
            q0 = q0 + e0 * e0
            s1 = s1 + e1
            q1 = q1 + e1 * e1
        t = (s0 * s0 - q0) + (s1 * s1 - q1)
        l0 = plsc.load_gather(lin_v, [r * _FPAD + lane])
        l1 = plsc.load_gather(lin_v, [r * _FPAD + 16 + lane])
        l1 = jnp.where(padmask, l1, 0.0)
        tvals_v[r, :] = 0.5 * t + l0 + l1
        return carry

    lax.fori_loop(0, _BPW, row_body, 0)

    for g in range(_BPW // 16):
        rows16 = g * 16 + lane
        acc = zero
        for d in range(16):
            acc = acc + plsc.load_gather(
                tvals_v, [rows16, jnp.full((16,), d, jnp.int32)])
        out_v[pl.ds(g * 16, 16)] = acc

    pltpu.sync_copy(out_v, out_hbm.at[w])


_fm_kernel = functools.partial(
    pl.kernel,
    out_type=jax.ShapeDtypeStruct((_NW, _BPW), jnp.float32),
    mesh=plsc.VectorSubcoreMesh(core_axis_name="c", subcore_axis_name="s"),
    scratch_types=[
        pltpu.VMEM((_F, _CHUNK), jnp.int32),
        pltpu.VMEM((_FPAD, _CHUNK), jnp.int32),
        pltpu.VMEM((_BPW * _F, _D), jnp.float32),
        pltpu.VMEM((_BPW * _FPAD,), jnp.float32),
        pltpu.VMEM((_BPW, 16), jnp.float32),
        pltpu.VMEM((_BPW,), jnp.float32),
        pltpu.SemaphoreType.DMA,
        pltpu.SemaphoreType.DMA,
    ],
    compiler_params=pltpu.CompilerParams(
        needs_layout_passes=False, use_tc_tiling_on_sc=False),
)(_fm_body)


def kernel(x, emb_table, linear_table, bias):
    offsets = jnp.asarray(_OFFSETS)
    idx = x + offsets[None, :]
    idxp = jnp.concatenate([idx, idx[:, : _FPAD - _F]], axis=1)
    idx3 = idx.reshape(_NW, _F, _CHUNK)
    idxp3 = idxp.reshape(_NW, _FPAD, _CHUNK)
    lin_flat = linear_table.reshape(-1)
    out = _fm_kernel(idx3, idxp3, emb_table, lin_flat)
    return out.reshape(_B, 1) + bias
